# baseline (speedup 1.0000x reference)
import functools

import jax
import jax.numpy as jnp
from jax import lax
from jax.experimental import pallas as pl
from jax.experimental.pallas import tpu as pltpu
from jax.experimental.pallas import tpu_sc as plsc

VOCAB = 1000000
EMBED_DIM = 64
BATCH = 4096
HIST = 200
N = BATCH * HIST

_info = plsc.get_sparse_core_info()
NC = _info.num_cores
NS = _info.num_subcores
NW = NC * NS
PER_W = N // NW

IDX_W = 128
GROUPS = 4
CHUNK = IDX_W * GROUPS
STEPS = PER_W // CHUNK
PAIRS = STEPS // 2

_mesh = plsc.VectorSubcoreMesh(core_axis_name="c", subcore_axis_name="s")


@functools.partial(
    pl.kernel,
    mesh=_mesh,
    out_type=jax.ShapeDtypeStruct((N, EMBED_DIM), jnp.float32),
    scratch_types=[
        pltpu.VMEM((PER_W,), jnp.int32),
        pltpu.VMEM((CHUNK, EMBED_DIM), jnp.float32),
        pltpu.VMEM((CHUNK, EMBED_DIM), jnp.float32),
        pltpu.SemaphoreType.DMA,
        pltpu.SemaphoreType.DMA,
    ],
    compiler_params=pltpu.CompilerParams(use_tc_tiling_on_sc=False),
)
def _gather_kernel(idx_hbm, table_hbm, out_hbm, idx_v, rows0, rows1, sem0, sem1):
    wid = lax.axis_index("s") * NC + lax.axis_index("c")
    base = wid * PER_W

    pltpu.sync_copy(idx_hbm.at[pl.ds(base, PER_W)], idx_v)

    rows = (rows0, rows1)
    sems = (sem0, sem1)

    def fire(c, b):
        for g in range(GROUPS):
            pltpu.async_copy(
                table_hbm.at[idx_v.at[pl.ds(c * CHUNK + g * IDX_W, IDX_W)]],
                rows[b].at[pl.ds(g * IDX_W, IDX_W), :],
                sems[b],
            )

    def drain(c, b):
        for g in range(GROUPS):
            pltpu.make_async_copy(
                table_hbm.at[idx_v.at[pl.ds(c * CHUNK + g * IDX_W, IDX_W)]],
                rows[b].at[pl.ds(g * IDX_W, IDX_W), :],
                sems[b],
            ).wait()

    def write(c, b):
        pltpu.sync_copy(rows[b], out_hbm.at[pl.ds(base + c * CHUNK, CHUNK), :])

    fire(0, 0)

    def pair(j, carry):
        c0 = 2 * j
        fire(c0 + 1, 1)
        drain(c0, 0)
        write(c0, 0)

        @pl.when(j < PAIRS - 1)
        def _():
            fire(c0 + 2, 0)

        drain(c0 + 1, 1)
        write(c0 + 1, 1)
        return carry

    lax.fori_loop(0, PAIRS, pair, 0)


def kernel(x, table):
    idx = jnp.minimum(jnp.maximum(x.reshape(N).astype(jnp.int32), 0), VOCAB - 1)
    out = _gather_kernel(idx, table)
    return out.reshape(BATCH, HIST, EMBED_DIM)


# speedup vs baseline: 1.0335x; 1.0335x over previous
import functools

import jax
import jax.numpy as jnp
from jax import lax
from jax.experimental import pallas as pl
from jax.experimental.pallas import tpu as pltpu
from jax.experimental.pallas import tpu_sc as plsc

VOCAB = 1000000
EMBED_DIM = 64
BATCH = 4096
HIST = 200
N = BATCH * HIST

_info = plsc.get_sparse_core_info()
NC = _info.num_cores
NS = _info.num_subcores
NW = NC * NS
PER_W = N // NW

IDX_W = 128
GROUPS = 4
CHUNK = IDX_W * GROUPS
STEPS = PER_W // CHUNK
PAIRS = STEPS // 2

_mesh = plsc.VectorSubcoreMesh(core_axis_name="c", subcore_axis_name="s")


@functools.partial(
    pl.kernel,
    mesh=_mesh,
    out_type=jax.ShapeDtypeStruct((N, EMBED_DIM), jnp.float32),
    scratch_types=[
        pltpu.VMEM((PER_W,), jnp.int32),
        pltpu.VMEM((CHUNK, EMBED_DIM), jnp.float32),
        pltpu.VMEM((CHUNK, EMBED_DIM), jnp.float32),
        pltpu.SemaphoreType.DMA,
        pltpu.SemaphoreType.DMA,
    ],
    compiler_params=pltpu.CompilerParams(use_tc_tiling_on_sc=False),
)
def _gather_kernel(idx_hbm, table_hbm, out_hbm, idx_v, rows0, rows1, sem0, sem1):
    wid = lax.axis_index("s") * NC + lax.axis_index("c")
    base = wid * PER_W

    pltpu.sync_copy(idx_hbm.at[pl.ds(base, PER_W)], idx_v)

    rows = (rows0, rows1)
    sems = (sem0, sem1)

    def fire(c, b):
        for g in range(GROUPS):
            pltpu.async_copy(
                table_hbm.at[idx_v.at[pl.ds(c * CHUNK + g * IDX_W, IDX_W)]],
                rows[b].at[pl.ds(g * IDX_W, IDX_W), :],
                sems[b],
            )

    def drain(c, b):
        for g in range(GROUPS):
            pltpu.make_async_copy(
                table_hbm.at[idx_v.at[pl.ds(c * CHUNK + g * IDX_W, IDX_W)]],
                rows[b].at[pl.ds(g * IDX_W, IDX_W), :],
                sems[b],
            ).wait()

    def write(c, b):
        pltpu.sync_copy(rows[b], out_hbm.at[pl.ds(base + c * CHUNK, CHUNK), :])

    fire(0, 0)

    def pair(j, carry):
        c0 = 2 * j
        fire(c0 + 1, 1)
        drain(c0, 0)
        write(c0, 0)

        @pl.when(j < PAIRS - 1)
        def _():
            fire(c0 + 2, 0)

        drain(c0 + 1, 1)
        write(c0 + 1, 1)
        return carry

    lax.fori_loop(0, PAIRS, pair, 0)


def kernel(x, table):
    idx = jnp.minimum(
        jnp.maximum(x.T.reshape(N).astype(jnp.int32), 0), VOCAB - 1
    )
    out = _gather_kernel(idx, table)
    return jnp.transpose(out.reshape(HIST, BATCH, EMBED_DIM), (1, 0, 2))
